# baseline (speedup 1.0000x reference)
import functools

import jax
import jax.numpy as jnp
from jax import lax
from jax.experimental import pallas as pl
from jax.experimental.pallas import tpu as pltpu

D = 128
BB = 32


@jax.jit
def _tc_permute(x, perm):
    B, S, _ = x.shape

    def body(x_ref, perm_ref, o_ref):
        iota = lax.broadcasted_iota(jnp.int32, (D, D), 0)
        onehot = (iota == perm_ref[...][None, :]).astype(jnp.float32)
        o_ref[...] = lax.dot_general(
            x_ref[...], onehot, (((2,), (0,)), ((), ())),
            preferred_element_type=jnp.float32)

    return pl.pallas_call(
        body,
        grid=(B // BB,),
        in_specs=[
            pl.BlockSpec((BB, S, D), lambda i: (i, 0, 0)),
            pl.BlockSpec((D,), lambda i: (0,)),
        ],
        out_specs=pl.BlockSpec((BB, S, D), lambda i: (i, 0, 0)),
        out_shape=jax.ShapeDtypeStruct((B, S, D), jnp.float32),
    )(x, perm)


def kernel(input, permutation):
    return _tc_permute(input, permutation.astype(jnp.int32))
